# baseline (speedup 1.0000x reference)
import jax
import jax.numpy as jnp
from jax import lax
from jax.experimental import pallas as pl
from jax.experimental.pallas import tpu as pltpu
from jax.experimental.pallas import tpu_sc as plsc

N = 100000
E = 3200000
K_ITERS = 10
ALPHA = 0.1
D = 8
NC = 2
NT = 16
NW = NC * NT
N_PAD = 102400
ROWS_PER_TILE = N_PAD // NT
RCHUNK = 256
NCHUNKS = ROWS_PER_TILE // RCHUNK
SUB = 128
SUBS = 16
SUPER = SUB * SUBS
SUPERS_PER_TILE = 50
EDGES_PER_TILE = SUPER * SUPERS_PER_TILE
E_PAD = EDGES_PER_TILE * NW
EROWS_PER_TILE = EDGES_PER_TILE // SUB

ROW_BLK = 800
GRID_N = N // ROW_BLK
GRID_NPAD = N_PAD // ROW_BLK

_SC_MESH = plsc.VectorSubcoreMesh(core_axis_name="c", subcore_axis_name="s")


def _mlp_body(x_ref, w1_ref, b1_ref, w2_ref, b2_ref, o_ref):
    h = jnp.dot(x_ref[...], w1_ref[...], preferred_element_type=jnp.float32)
    h = jnp.maximum(h + b1_ref[...], 0.0)
    o = jnp.dot(h, w2_ref[...], preferred_element_type=jnp.float32)
    o_ref[...] = o + b2_ref[...]


def _mlp(x, W1, b1r, W2p, b2p):
    return pl.pallas_call(
        _mlp_body,
        grid=(GRID_N,),
        in_specs=[
            pl.BlockSpec((ROW_BLK, 1433), lambda i: (i, 0)),
            pl.BlockSpec((1433, 64), lambda i: (0, 0)),
            pl.BlockSpec((1, 64), lambda i: (0, 0)),
            pl.BlockSpec((64, D), lambda i: (0, 0)),
            pl.BlockSpec((1, D), lambda i: (0, 0)),
        ],
        out_specs=pl.BlockSpec((ROW_BLK, D), lambda i: (i, 0)),
        out_shape=jax.ShapeDtypeStruct((N, D), jnp.float32),
    )(x, W1, b1r, W2p, b2p)


def _deg_body(dst_hbm, ones_hbm, zeros_hbm, deg_hbm,
              idx_v, ones_v, stage_v, deg_sh, ssem):
    cid = lax.axis_index("c")
    sid = lax.axis_index("s")
    wid = sid * NC + cid
    row0 = sid * ROWS_PER_TILE
    erow0 = wid * EROWS_PER_TILE

    pltpu.sync_copy(ones_hbm, ones_v)
    pltpu.sync_copy(zeros_hbm, stage_v)

    @pl.loop(0, NCHUNKS)
    def _zero(c):
        pltpu.sync_copy(stage_v, deg_sh.at[pl.ds(row0 + c * RCHUNK, RCHUNK), :])

    plsc.subcore_barrier()

    @pl.loop(0, SUPERS_PER_TILE)
    def _scatter(sc):
        pltpu.sync_copy(dst_hbm.at[pl.ds(erow0 + sc * SUBS, SUBS), :], idx_v)
        handles = []
        for j in range(SUBS):
            handles.append(
                pltpu.async_copy(ones_v, deg_sh.at[idx_v.at[j]], ssem,
                                 add=True))
        for h in handles:
            h.wait()

    plsc.subcore_barrier()

    @pl.loop(0, NCHUNKS)
    def _dump(c):
        r = row0 + c * RCHUNK
        pltpu.sync_copy(deg_sh.at[pl.ds(r, RCHUNK), :], stage_v)
        pltpu.sync_copy(stage_v, deg_hbm.at[cid, pl.ds(r, RCHUNK), :])


def _degrees(dst2d, ones8, zeros8):
    return pl.kernel(
        _deg_body,
        out_type=jax.ShapeDtypeStruct((NC, N_PAD, D), jnp.float32),
        mesh=_SC_MESH,
        compiler_params=pltpu.CompilerParams(use_tc_tiling_on_sc=False),
        scratch_types=[
            pltpu.VMEM((SUBS, SUB), jnp.int32),
            pltpu.VMEM((SUB, D), jnp.float32),
            pltpu.VMEM((RCHUNK, D), jnp.float32),
            pltpu.VMEM_SHARED((N_PAD, D), jnp.float32),
            pltpu.SemaphoreType.DMA,
        ],
    )(dst2d, ones8, zeros8)


def _pre_body(deg_ref, h0_ref, c1_ref, z_ref, g0_ref, dsq_ref):
    deg = deg_ref[0] + deg_ref[1] + 1.0
    dinv = lax.rsqrt(deg)
    h0 = h0_ref[...]
    c1_ref[...] = (1.0 - ALPHA) * dinv * dinv
    z_ref[...] = ALPHA * dinv * h0
    g0_ref[...] = dinv * h0
    dsq_ref[...] = jnp.sqrt(deg)


def _precompute(deg2, h0p):
    return pl.pallas_call(
        _pre_body,
        grid=(GRID_NPAD,),
        in_specs=[
            pl.BlockSpec((NC, ROW_BLK, D), lambda i: (0, i, 0)),
            pl.BlockSpec((ROW_BLK, D), lambda i: (i, 0)),
        ],
        out_specs=[pl.BlockSpec((ROW_BLK, D), lambda i: (i, 0))] * 4,
        out_shape=[jax.ShapeDtypeStruct((N_PAD, D), jnp.float32)] * 4,
    )(deg2, h0p)


def _round_body(src_hbm, dst_hbm, g_hbm, zeros_hbm, s_hbm,
                src_v, dst_v, rows_v, stage_v, s_sh, gsem, ssem):
    cid = lax.axis_index("c")
    sid = lax.axis_index("s")
    wid = sid * NC + cid
    row0 = sid * ROWS_PER_TILE
    erow0 = wid * EROWS_PER_TILE

    pltpu.sync_copy(zeros_hbm, stage_v)

    @pl.loop(0, NCHUNKS)
    def _zero(c):
        pltpu.sync_copy(stage_v, s_sh.at[pl.ds(row0 + c * RCHUNK, RCHUNK), :])

    plsc.subcore_barrier()

    @pl.loop(0, SUPERS_PER_TILE)
    def _scatter(sc):
        er = erow0 + sc * SUBS
        pltpu.sync_copy(src_hbm.at[pl.ds(er, SUBS), :], src_v)
        pltpu.sync_copy(dst_hbm.at[pl.ds(er, SUBS), :], dst_v)
        gh = []
        for j in range(SUBS):
            gh.append(pltpu.async_copy(
                g_hbm.at[src_v.at[j]], rows_v.at[j], gsem))
        for h in gh:
            h.wait()
        sh = []
        for j in range(SUBS):
            sh.append(pltpu.async_copy(
                rows_v.at[j], s_sh.at[dst_v.at[j]], ssem, add=True))
        for h in sh:
            h.wait()

    plsc.subcore_barrier()

    @pl.loop(0, NCHUNKS)
    def _dump(c):
        r = row0 + c * RCHUNK
        pltpu.sync_copy(s_sh.at[pl.ds(r, RCHUNK), :], stage_v)
        pltpu.sync_copy(stage_v, s_hbm.at[cid, pl.ds(r, RCHUNK), :])


def _scatter_round(src2d, dst2d, g, zeros8):
    return pl.kernel(
        _round_body,
        out_type=jax.ShapeDtypeStruct((NC, N_PAD, D), jnp.float32),
        mesh=_SC_MESH,
        compiler_params=pltpu.CompilerParams(use_tc_tiling_on_sc=False),
        scratch_types=[
            pltpu.VMEM((SUBS, SUB), jnp.int32),
            pltpu.VMEM((SUBS, SUB), jnp.int32),
            pltpu.VMEM((SUBS, SUB, D), jnp.float32),
            pltpu.VMEM((RCHUNK, D), jnp.float32),
            pltpu.VMEM_SHARED((N_PAD, D), jnp.float32),
            pltpu.SemaphoreType.DMA,
            pltpu.SemaphoreType.DMA,
        ],
    )(src2d, dst2d, g, zeros8)


def _dense_body(s_ref, g_ref, c1_ref, z_ref, o_ref):
    s = s_ref[0] + s_ref[1]
    o_ref[...] = c1_ref[...] * (s + g_ref[...]) + z_ref[...]


def _dense_update(s2, g, c1e, z):
    return pl.pallas_call(
        _dense_body,
        grid=(GRID_NPAD,),
        in_specs=[
            pl.BlockSpec((NC, ROW_BLK, D), lambda i: (0, i, 0)),
            pl.BlockSpec((ROW_BLK, D), lambda i: (i, 0)),
            pl.BlockSpec((ROW_BLK, D), lambda i: (i, 0)),
            pl.BlockSpec((ROW_BLK, D), lambda i: (i, 0)),
        ],
        out_specs=pl.BlockSpec((ROW_BLK, D), lambda i: (i, 0)),
        out_shape=jax.ShapeDtypeStruct((N_PAD, D), jnp.float32),
    )(s2, g, c1e, z)


def _final_body(g_ref, dsq_ref, o_ref):
    h = g_ref[...] * dsq_ref[...]
    col = lax.broadcasted_iota(jnp.int32, (ROW_BLK, D), 1)
    mask = col < 7
    hm = jnp.where(mask, h, -3e38)
    m = jnp.max(hm, axis=1, keepdims=True)
    e = jnp.where(mask, jnp.exp(h - m), 0.0)
    ls = h - m - jnp.log(jnp.sum(e, axis=1, keepdims=True))
    o_ref[...] = ls[:, :7]


def _finalize(g10, dsq):
    return pl.pallas_call(
        _final_body,
        grid=(GRID_N,),
        in_specs=[pl.BlockSpec((ROW_BLK, D), lambda i: (i, 0))] * 2,
        out_specs=pl.BlockSpec((ROW_BLK, 7), lambda i: (i, 0)),
        out_shape=jax.ShapeDtypeStruct((N, 7), jnp.float32),
    )(g10, dsq)


def kernel(x, edge_index, W1, b1, W2, b2):
    pad = jnp.full((E_PAD - E,), N, jnp.int32)
    src2d = jnp.concatenate([edge_index[0], pad]).reshape(E_PAD // SUB, SUB)
    dst2d = jnp.concatenate([edge_index[1], pad]).reshape(E_PAD // SUB, SUB)

    W2p = jnp.zeros((64, D), jnp.float32).at[:, :7].set(W2)
    b2p = jnp.zeros((1, D), jnp.float32).at[0, :7].set(b2)
    ones8 = jnp.ones((SUB, D), jnp.float32)
    zeros8 = jnp.zeros((RCHUNK, D), jnp.float32)

    h0 = _mlp(x, W1, b1.reshape(1, 64), W2p, b2p)
    h0p = jnp.pad(h0, ((0, N_PAD - N), (0, 0)))

    deg2 = _degrees(dst2d, ones8, zeros8)
    c1e, z, g, dsq = _precompute(deg2, h0p)
    for _ in range(K_ITERS):
        s2 = _scatter_round(src2d, dst2d, g, zeros8)
        g = _dense_update(s2, g, c1e, z)
    return _finalize(g, dsq)


# speedup vs baseline: 1.6937x; 1.6937x over previous
import jax
import jax.numpy as jnp
from jax import lax
from jax.experimental import pallas as pl
from jax.experimental.pallas import tpu as pltpu
from jax.experimental.pallas import tpu_sc as plsc

N = 100000
E = 3200000
K_ITERS = 10
ALPHA = 0.1
D = 8
NC = 2
NT = 16
NW = NC * NT
N_PAD = 102400
ROWS_PER_TILE = N_PAD // NT
RCHUNK = 256
NCHUNKS = ROWS_PER_TILE // RCHUNK
SUPER = 4096
SUPERS_PER_TILE = 25
EDGES_PER_TILE = SUPER * SUPERS_PER_TILE
E_PAD = EDGES_PER_TILE * NW

ROW_BLK = 800
GRID_N = N // ROW_BLK
GRID_NPAD = N_PAD // ROW_BLK

_SC_MESH = plsc.VectorSubcoreMesh(core_axis_name="c", subcore_axis_name="s")


def _mlp_body(x_ref, w1_ref, b1_ref, w2_ref, b2_ref, o_ref):
    h = jnp.dot(x_ref[...], w1_ref[...], preferred_element_type=jnp.float32)
    h = jnp.maximum(h + b1_ref[...], 0.0)
    o = jnp.dot(h, w2_ref[...], preferred_element_type=jnp.float32)
    o_ref[...] = o + b2_ref[...]


def _mlp(x, W1, b1r, W2p, b2p):
    return pl.pallas_call(
        _mlp_body,
        grid=(GRID_N,),
        in_specs=[
            pl.BlockSpec((ROW_BLK, 1433), lambda i: (i, 0)),
            pl.BlockSpec((1433, 64), lambda i: (0, 0)),
            pl.BlockSpec((1, 64), lambda i: (0, 0)),
            pl.BlockSpec((64, D), lambda i: (0, 0)),
            pl.BlockSpec((1, D), lambda i: (0, 0)),
        ],
        out_specs=pl.BlockSpec((ROW_BLK, D), lambda i: (i, 0)),
        out_shape=jax.ShapeDtypeStruct((N, D), jnp.float32),
    )(x, W1, b1r, W2p, b2p)


def _deg_body(dst_hbm, ones_hbm, zeros_hbm, deg_hbm,
              idx_v, ones_v, stage_v, deg_sh, ssem):
    cid = lax.axis_index("c")
    sid = lax.axis_index("s")
    wid = sid * NC + cid
    row0 = sid * ROWS_PER_TILE
    e0 = wid * EDGES_PER_TILE

    pltpu.sync_copy(ones_hbm, ones_v)
    pltpu.sync_copy(zeros_hbm, stage_v)

    @pl.loop(0, NCHUNKS)
    def _zero(c):
        pltpu.sync_copy(stage_v, deg_sh.at[pl.ds(row0 + c * RCHUNK, RCHUNK), :])

    plsc.subcore_barrier()

    @pl.loop(0, SUPERS_PER_TILE)
    def _scatter(sc):
        pltpu.sync_copy(dst_hbm.at[pl.ds(e0 + sc * SUPER, SUPER)], idx_v)
        pltpu.async_copy(ones_v, deg_sh.at[idx_v], ssem, add=True).wait()

    plsc.subcore_barrier()

    @pl.loop(0, NCHUNKS)
    def _dump(c):
        r = row0 + c * RCHUNK
        pltpu.sync_copy(deg_sh.at[pl.ds(r, RCHUNK), :], stage_v)
        pltpu.sync_copy(stage_v, deg_hbm.at[cid, pl.ds(r, RCHUNK), :])


def _degrees(dst1d, ones8, zeros8):
    return pl.kernel(
        _deg_body,
        out_type=jax.ShapeDtypeStruct((NC, N_PAD, D), jnp.float32),
        mesh=_SC_MESH,
        compiler_params=pltpu.CompilerParams(use_tc_tiling_on_sc=False),
        scratch_types=[
            pltpu.VMEM((SUPER,), jnp.int32),
            pltpu.VMEM((SUPER, D), jnp.float32),
            pltpu.VMEM((RCHUNK, D), jnp.float32),
            pltpu.VMEM_SHARED((N_PAD, D), jnp.float32),
            pltpu.SemaphoreType.DMA,
        ],
    )(dst1d, ones8, zeros8)


def _pre_body(deg_ref, h0_ref, c1_ref, z_ref, g0_ref, dsq_ref):
    deg = deg_ref[0] + deg_ref[1] + 1.0
    dinv = lax.rsqrt(deg)
    h0 = h0_ref[...]
    c1_ref[...] = (1.0 - ALPHA) * dinv * dinv
    z_ref[...] = ALPHA * dinv * h0
    g0_ref[...] = dinv * h0
    dsq_ref[...] = jnp.sqrt(deg)


def _precompute(deg2, h0p):
    return pl.pallas_call(
        _pre_body,
        grid=(GRID_NPAD,),
        in_specs=[
            pl.BlockSpec((NC, ROW_BLK, D), lambda i: (0, i, 0)),
            pl.BlockSpec((ROW_BLK, D), lambda i: (i, 0)),
        ],
        out_specs=[pl.BlockSpec((ROW_BLK, D), lambda i: (i, 0))] * 4,
        out_shape=[jax.ShapeDtypeStruct((N_PAD, D), jnp.float32)] * 4,
    )(deg2, h0p)


def _round_body(src_hbm, dst_hbm, g_hbm, zeros_hbm, s_hbm,
                src_v, dst_v, rows_v, stage_v, s_sh, gsem, ssem):
    cid = lax.axis_index("c")
    sid = lax.axis_index("s")
    wid = sid * NC + cid
    row0 = sid * ROWS_PER_TILE
    e0 = wid * EDGES_PER_TILE

    pltpu.sync_copy(zeros_hbm, stage_v)

    @pl.loop(0, NCHUNKS)
    def _zero(c):
        pltpu.sync_copy(stage_v, s_sh.at[pl.ds(row0 + c * RCHUNK, RCHUNK), :])

    plsc.subcore_barrier()

    @pl.loop(0, SUPERS_PER_TILE)
    def _scatter(sc):
        e = e0 + sc * SUPER
        pltpu.sync_copy(src_hbm.at[pl.ds(e, SUPER)], src_v)
        pltpu.sync_copy(dst_hbm.at[pl.ds(e, SUPER)], dst_v)
        pltpu.async_copy(g_hbm.at[src_v], rows_v, gsem).wait()
        pltpu.async_copy(rows_v, s_sh.at[dst_v], ssem, add=True).wait()

    plsc.subcore_barrier()

    @pl.loop(0, NCHUNKS)
    def _dump(c):
        r = row0 + c * RCHUNK
        pltpu.sync_copy(s_sh.at[pl.ds(r, RCHUNK), :], stage_v)
        pltpu.sync_copy(stage_v, s_hbm.at[cid, pl.ds(r, RCHUNK), :])


def _scatter_round(src1d, dst1d, g, zeros8):
    return pl.kernel(
        _round_body,
        out_type=jax.ShapeDtypeStruct((NC, N_PAD, D), jnp.float32),
        mesh=_SC_MESH,
        compiler_params=pltpu.CompilerParams(use_tc_tiling_on_sc=False),
        scratch_types=[
            pltpu.VMEM((SUPER,), jnp.int32),
            pltpu.VMEM((SUPER,), jnp.int32),
            pltpu.VMEM((SUPER, D), jnp.float32),
            pltpu.VMEM((RCHUNK, D), jnp.float32),
            pltpu.VMEM_SHARED((N_PAD, D), jnp.float32),
            pltpu.SemaphoreType.DMA,
            pltpu.SemaphoreType.DMA,
        ],
    )(src1d, dst1d, g, zeros8)


def _dense_body(s_ref, g_ref, c1_ref, z_ref, o_ref):
    s = s_ref[0] + s_ref[1]
    o_ref[...] = c1_ref[...] * (s + g_ref[...]) + z_ref[...]


def _dense_update(s2, g, c1e, z):
    return pl.pallas_call(
        _dense_body,
        grid=(GRID_NPAD,),
        in_specs=[
            pl.BlockSpec((NC, ROW_BLK, D), lambda i: (0, i, 0)),
            pl.BlockSpec((ROW_BLK, D), lambda i: (i, 0)),
            pl.BlockSpec((ROW_BLK, D), lambda i: (i, 0)),
            pl.BlockSpec((ROW_BLK, D), lambda i: (i, 0)),
        ],
        out_specs=pl.BlockSpec((ROW_BLK, D), lambda i: (i, 0)),
        out_shape=jax.ShapeDtypeStruct((N_PAD, D), jnp.float32),
    )(s2, g, c1e, z)


def _final_body(g_ref, dsq_ref, o_ref):
    h = g_ref[...] * dsq_ref[...]
    col = lax.broadcasted_iota(jnp.int32, (ROW_BLK, D), 1)
    mask = col < 7
    hm = jnp.where(mask, h, -3e38)
    m = jnp.max(hm, axis=1, keepdims=True)
    e = jnp.where(mask, jnp.exp(h - m), 0.0)
    ls = h - m - jnp.log(jnp.sum(e, axis=1, keepdims=True))
    o_ref[...] = ls[:, :7]


def _finalize(g10, dsq):
    return pl.pallas_call(
        _final_body,
        grid=(GRID_N,),
        in_specs=[pl.BlockSpec((ROW_BLK, D), lambda i: (i, 0))] * 2,
        out_specs=pl.BlockSpec((ROW_BLK, 7), lambda i: (i, 0)),
        out_shape=jax.ShapeDtypeStruct((N, 7), jnp.float32),
    )(g10, dsq)


def kernel(x, edge_index, W1, b1, W2, b2):
    pad = N + jnp.arange(E_PAD - E, dtype=jnp.int32) % (N_PAD - N)
    src1d = jnp.concatenate([edge_index[0], pad])
    dst1d = jnp.concatenate([edge_index[1], pad])

    W2p = jnp.zeros((64, D), jnp.float32).at[:, :7].set(W2)
    b2p = jnp.zeros((1, D), jnp.float32).at[0, :7].set(b2)
    ones8 = jnp.ones((SUPER, D), jnp.float32)
    zeros8 = jnp.zeros((RCHUNK, D), jnp.float32)

    h0 = _mlp(x, W1, b1.reshape(1, 64), W2p, b2p)
    h0p = jnp.pad(h0, ((0, N_PAD - N), (0, 0)))

    deg2 = _degrees(dst1d, ones8, zeros8)
    c1e, z, g, dsq = _precompute(deg2, h0p)
    for _ in range(K_ITERS):
        s2 = _scatter_round(src1d, dst1d, g, zeros8)
        g = _dense_update(s2, g, c1e, z)
    return _finalize(g, dsq)


# speedup vs baseline: 1.8500x; 1.0923x over previous
import jax
import jax.numpy as jnp
from jax import lax
from jax.experimental import pallas as pl
from jax.experimental.pallas import tpu as pltpu
from jax.experimental.pallas import tpu_sc as plsc

N = 100000
E = 3200000
K_ITERS = 10
ALPHA = 0.1
D = 8
NC = 2
NT = 16
NW = NC * NT
N_PAD = 102400
ROWS_PER_TILE = N_PAD // NT
RCHUNK = 256
NCHUNKS = ROWS_PER_TILE // RCHUNK
SUPER = 2048
SUPERS_PER_TILE = 50
EDGES_PER_TILE = SUPER * SUPERS_PER_TILE
E_PAD = EDGES_PER_TILE * NW

ROW_BLK = 800
GRID_N = N // ROW_BLK
GRID_NPAD = N_PAD // ROW_BLK

_SC_MESH = plsc.VectorSubcoreMesh(core_axis_name="c", subcore_axis_name="s")


def _mlp_body(x_ref, w1_ref, b1_ref, w2_ref, b2_ref, o_ref):
    h = jnp.dot(x_ref[...], w1_ref[...], preferred_element_type=jnp.float32)
    h = jnp.maximum(h + b1_ref[...], 0.0)
    o = jnp.dot(h, w2_ref[...], preferred_element_type=jnp.float32)
    o_ref[...] = o + b2_ref[...]


def _mlp(x, W1, b1r, W2p, b2p):
    return pl.pallas_call(
        _mlp_body,
        grid=(GRID_N,),
        in_specs=[
            pl.BlockSpec((ROW_BLK, 1433), lambda i: (i, 0)),
            pl.BlockSpec((1433, 64), lambda i: (0, 0)),
            pl.BlockSpec((1, 64), lambda i: (0, 0)),
            pl.BlockSpec((64, D), lambda i: (0, 0)),
            pl.BlockSpec((1, D), lambda i: (0, 0)),
        ],
        out_specs=pl.BlockSpec((ROW_BLK, D), lambda i: (i, 0)),
        out_shape=jax.ShapeDtypeStruct((N, D), jnp.float32),
    )(x, W1, b1r, W2p, b2p)


def _deg_body(dst_hbm, ones_hbm, zeros_hbm, deg_hbm,
              idx_v, ones_v, stage_v, deg_sh, ssem):
    cid = lax.axis_index("c")
    sid = lax.axis_index("s")
    wid = sid * NC + cid
    row0 = sid * ROWS_PER_TILE
    e0 = wid * EDGES_PER_TILE

    pltpu.sync_copy(ones_hbm, ones_v)
    pltpu.sync_copy(zeros_hbm, stage_v)

    @pl.loop(0, NCHUNKS)
    def _zero(c):
        pltpu.sync_copy(stage_v, deg_sh.at[pl.ds(row0 + c * RCHUNK, RCHUNK), :])

    plsc.subcore_barrier()

    @pl.loop(0, SUPERS_PER_TILE)
    def _scatter(sc):
        pltpu.sync_copy(dst_hbm.at[pl.ds(e0 + sc * SUPER, SUPER)], idx_v)
        pltpu.async_copy(ones_v, deg_sh.at[idx_v], ssem, add=True).wait()

    plsc.subcore_barrier()

    @pl.loop(0, NCHUNKS)
    def _dump(c):
        r = row0 + c * RCHUNK
        pltpu.sync_copy(deg_sh.at[pl.ds(r, RCHUNK), :], stage_v)
        pltpu.sync_copy(stage_v, deg_hbm.at[cid, pl.ds(r, RCHUNK), :])


def _degrees(dst1d, ones8, zeros8):
    return pl.kernel(
        _deg_body,
        out_type=jax.ShapeDtypeStruct((NC, N_PAD, D), jnp.float32),
        mesh=_SC_MESH,
        compiler_params=pltpu.CompilerParams(use_tc_tiling_on_sc=False),
        scratch_types=[
            pltpu.VMEM((SUPER,), jnp.int32),
            pltpu.VMEM((SUPER, D), jnp.float32),
            pltpu.VMEM((RCHUNK, D), jnp.float32),
            pltpu.VMEM_SHARED((N_PAD, D), jnp.float32),
            pltpu.SemaphoreType.DMA,
        ],
    )(dst1d, ones8, zeros8)


def _pre_body(deg_ref, h0_ref, c1_ref, z_ref, g0_ref, dsq_ref):
    deg = deg_ref[0] + deg_ref[1] + 1.0
    dinv = lax.rsqrt(deg)
    h0 = h0_ref[...]
    c1_ref[...] = (1.0 - ALPHA) * dinv * dinv
    z_ref[...] = ALPHA * dinv * h0
    g0_ref[...] = dinv * h0
    dsq_ref[...] = jnp.sqrt(deg)


def _precompute(deg2, h0p):
    return pl.pallas_call(
        _pre_body,
        grid=(GRID_NPAD,),
        in_specs=[
            pl.BlockSpec((NC, ROW_BLK, D), lambda i: (0, i, 0)),
            pl.BlockSpec((ROW_BLK, D), lambda i: (i, 0)),
        ],
        out_specs=[pl.BlockSpec((ROW_BLK, D), lambda i: (i, 0))] * 4,
        out_shape=[jax.ShapeDtypeStruct((N_PAD, D), jnp.float32)] * 4,
    )(deg2, h0p)


def _round_body(src_hbm, dst_hbm, g_hbm, zeros_hbm, s_hbm,
                src_v, dst_v, rows_v, stage_v, s_sh,
                gsem0, gsem1, ssem0, ssem1, isem0, isem1, jsem0, jsem1):
    cid = lax.axis_index("c")
    sid = lax.axis_index("s")
    wid = sid * NC + cid
    row0 = sid * ROWS_PER_TILE
    e0 = wid * EDGES_PER_TILE
    gsem = (gsem0, gsem1)
    ssem = (ssem0, ssem1)
    isem = (isem0, isem1)
    jsem = (jsem0, jsem1)
    NS = SUPERS_PER_TILE

    pltpu.sync_copy(zeros_hbm, stage_v)

    @pl.loop(0, NCHUNKS)
    def _zero(c):
        pltpu.sync_copy(stage_v, s_sh.at[pl.ds(row0 + c * RCHUNK, RCHUNK), :])

    plsc.subcore_barrier()

    def src_slice(sc):
        return src_hbm.at[pl.ds(e0 + sc * SUPER, SUPER)]

    def dst_slice(sc):
        return dst_hbm.at[pl.ds(e0 + sc * SUPER, SUPER)]

    def fire_g(sc, p):
        return pltpu.async_copy(g_hbm.at[src_v.at[p]], rows_v.at[p], gsem[p])

    def fire_s(sc, p):
        return pltpu.async_copy(rows_v.at[p], s_sh.at[dst_v.at[p]], ssem[p],
                                add=True)

    def drain_g(p):
        pltpu.make_async_copy(g_hbm.at[src_v.at[p]], rows_v.at[p],
                              gsem[p]).wait()

    def drain_s(p):
        pltpu.make_async_copy(rows_v.at[p], s_sh.at[dst_v.at[p]],
                              ssem[p]).wait()

    pltpu.sync_copy(src_slice(0), src_v.at[0])
    pltpu.sync_copy(dst_slice(0), dst_v.at[0])
    fire_g(0, 0)
    pltpu.async_copy(src_slice(1), src_v.at[1], jsem[1])
    pltpu.async_copy(dst_slice(1), dst_v.at[1], isem[1])

    @pl.loop(0, NS // 2)
    def _pair(t):
        for p in (0, 1):
            sc = t * 2 + p
            q = 1 - p

            if p == 0:
                @pl.when(sc > 0)
                def _():
                    drain_s(q)
                    pltpu.async_copy(dst_slice(sc + 1), dst_v.at[q], isem[q])
            else:
                drain_s(q)
                @pl.when(sc + 1 < NS)
                def _():
                    pltpu.async_copy(dst_slice(sc + 1), dst_v.at[q], isem[q])

            drain_g(p)

            @pl.when(sc + 2 < NS)
            def _():
                pltpu.async_copy(src_slice(sc + 2), src_v.at[p], jsem[p])

            fire_s(sc, p)

            if p == 0:
                pltpu.make_async_copy(src_slice(sc + 1), src_v.at[q],
                                      jsem[q]).wait()
                pltpu.make_async_copy(dst_slice(sc + 1), dst_v.at[q],
                                      isem[q]).wait()
                fire_g(sc + 1, q)
            else:
                @pl.when(sc + 1 < NS)
                def _():
                    pltpu.make_async_copy(src_slice(sc + 1), src_v.at[q],
                                          jsem[q]).wait()
                    pltpu.make_async_copy(dst_slice(sc + 1), dst_v.at[q],
                                          isem[q]).wait()
                    fire_g(sc + 1, q)

    drain_s(1)

    plsc.subcore_barrier()

    @pl.loop(0, NCHUNKS)
    def _dump(c):
        r = row0 + c * RCHUNK
        pltpu.sync_copy(s_sh.at[pl.ds(r, RCHUNK), :], stage_v)
        pltpu.sync_copy(stage_v, s_hbm.at[cid, pl.ds(r, RCHUNK), :])


def _scatter_round(src1d, dst1d, g, zeros8):
    return pl.kernel(
        _round_body,
        out_type=jax.ShapeDtypeStruct((NC, N_PAD, D), jnp.float32),
        mesh=_SC_MESH,
        compiler_params=pltpu.CompilerParams(use_tc_tiling_on_sc=False),
        scratch_types=[
            pltpu.VMEM((2, SUPER), jnp.int32),
            pltpu.VMEM((2, SUPER), jnp.int32),
            pltpu.VMEM((2, SUPER, D), jnp.float32),
            pltpu.VMEM((RCHUNK, D), jnp.float32),
            pltpu.VMEM_SHARED((N_PAD, D), jnp.float32),
        ] + [pltpu.SemaphoreType.DMA] * 8,
    )(src1d, dst1d, g, zeros8)


def _dense_body(s_ref, g_ref, c1_ref, z_ref, o_ref):
    s = s_ref[0] + s_ref[1]
    o_ref[...] = c1_ref[...] * (s + g_ref[...]) + z_ref[...]


def _dense_update(s2, g, c1e, z):
    return pl.pallas_call(
        _dense_body,
        grid=(GRID_NPAD,),
        in_specs=[
            pl.BlockSpec((NC, ROW_BLK, D), lambda i: (0, i, 0)),
            pl.BlockSpec((ROW_BLK, D), lambda i: (i, 0)),
            pl.BlockSpec((ROW_BLK, D), lambda i: (i, 0)),
            pl.BlockSpec((ROW_BLK, D), lambda i: (i, 0)),
        ],
        out_specs=pl.BlockSpec((ROW_BLK, D), lambda i: (i, 0)),
        out_shape=jax.ShapeDtypeStruct((N_PAD, D), jnp.float32),
    )(s2, g, c1e, z)


def _final_body(g_ref, dsq_ref, o_ref):
    h = g_ref[...] * dsq_ref[...]
    col = lax.broadcasted_iota(jnp.int32, (ROW_BLK, D), 1)
    mask = col < 7
    hm = jnp.where(mask, h, -3e38)
    m = jnp.max(hm, axis=1, keepdims=True)
    e = jnp.where(mask, jnp.exp(h - m), 0.0)
    ls = h - m - jnp.log(jnp.sum(e, axis=1, keepdims=True))
    o_ref[...] = ls[:, :7]


def _finalize(g10, dsq):
    return pl.pallas_call(
        _final_body,
        grid=(GRID_N,),
        in_specs=[pl.BlockSpec((ROW_BLK, D), lambda i: (i, 0))] * 2,
        out_specs=pl.BlockSpec((ROW_BLK, 7), lambda i: (i, 0)),
        out_shape=jax.ShapeDtypeStruct((N, 7), jnp.float32),
    )(g10, dsq)


def kernel(x, edge_index, W1, b1, W2, b2):
    pad = N + jnp.arange(E_PAD - E, dtype=jnp.int32) % (N_PAD - N)
    src1d = jnp.concatenate([edge_index[0], pad])
    dst1d = jnp.concatenate([edge_index[1], pad])

    W2p = jnp.zeros((64, D), jnp.float32).at[:, :7].set(W2)
    b2p = jnp.zeros((1, D), jnp.float32).at[0, :7].set(b2)
    ones8 = jnp.ones((SUPER, D), jnp.float32)
    zeros8 = jnp.zeros((RCHUNK, D), jnp.float32)

    h0 = _mlp(x, W1, b1.reshape(1, 64), W2p, b2p)
    h0p = jnp.pad(h0, ((0, N_PAD - N), (0, 0)))

    deg2 = _degrees(dst1d, ones8, zeros8)
    c1e, z, g, dsq = _precompute(deg2, h0p)
    for _ in range(K_ITERS):
        s2 = _scatter_round(src1d, dst1d, g, zeros8)
        g = _dense_update(s2, g, c1e, z)
    return _finalize(g, dsq)


# speedup vs baseline: 1.8528x; 1.0015x over previous
import jax
import jax.numpy as jnp
from jax import lax
from jax.experimental import pallas as pl
from jax.experimental.pallas import tpu as pltpu
from jax.experimental.pallas import tpu_sc as plsc

N = 100000
E = 3200000
K_ITERS = 10
ALPHA = 0.1
D = 8
NC = 2
NT = 16
NW = NC * NT
N_PAD = 102400
ROWS_PER_TILE = N_PAD // NT
RCHUNK = 256
NCHUNKS = ROWS_PER_TILE // RCHUNK
SUPER = 2048
SUPERS_PER_TILE = 50
EDGES_PER_TILE = SUPER * SUPERS_PER_TILE
E_PAD = EDGES_PER_TILE * NW

ROW_BLK = 800
GRID_N = N // ROW_BLK
GRID_NPAD = N_PAD // ROW_BLK

_SC_MESH = plsc.VectorSubcoreMesh(core_axis_name="c", subcore_axis_name="s")


def _mlp_body(x_ref, w1_ref, b1_ref, w2_ref, b2_ref, o_ref):
    h = jnp.dot(x_ref[...], w1_ref[...], preferred_element_type=jnp.float32)
    h = jnp.maximum(h + b1_ref[...], 0.0)
    o = jnp.dot(h, w2_ref[...], preferred_element_type=jnp.float32)
    o_ref[...] = o + b2_ref[...]


def _mlp(x, W1, b1r, W2p, b2p):
    return pl.pallas_call(
        _mlp_body,
        grid=(GRID_N,),
        in_specs=[
            pl.BlockSpec((ROW_BLK, 1433), lambda i: (i, 0)),
            pl.BlockSpec((1433, 64), lambda i: (0, 0)),
            pl.BlockSpec((1, 64), lambda i: (0, 0)),
            pl.BlockSpec((64, D), lambda i: (0, 0)),
            pl.BlockSpec((1, D), lambda i: (0, 0)),
        ],
        out_specs=pl.BlockSpec((ROW_BLK, D), lambda i: (i, 0)),
        out_shape=jax.ShapeDtypeStruct((N, D), jnp.float32),
    )(x, W1, b1r, W2p, b2p)


def _deg_body(dst_hbm, ones_hbm, zeros_hbm, deg0_hbm, deg1_hbm,
              idx_v, ones_v, stage_v, deg_sh, ssem):
    cid = lax.axis_index("c")
    sid = lax.axis_index("s")
    wid = sid * NC + cid
    row0 = sid * ROWS_PER_TILE
    e0 = wid * EDGES_PER_TILE

    pltpu.sync_copy(ones_hbm, ones_v)
    pltpu.sync_copy(zeros_hbm, stage_v)

    @pl.loop(0, NCHUNKS)
    def _zero(c):
        pltpu.sync_copy(stage_v, deg_sh.at[pl.ds(row0 + c * RCHUNK, RCHUNK), :])

    plsc.subcore_barrier()

    @pl.loop(0, SUPERS_PER_TILE)
    def _scatter(sc):
        pltpu.sync_copy(dst_hbm.at[pl.ds(e0 + sc * SUPER, SUPER)], idx_v)
        pltpu.async_copy(ones_v, deg_sh.at[idx_v], ssem, add=True).wait()

    plsc.subcore_barrier()

    @pl.loop(0, NCHUNKS)
    def _dump(c):
        r = row0 + c * RCHUNK
        pltpu.sync_copy(deg_sh.at[pl.ds(r, RCHUNK), :], stage_v)

        @pl.when(cid == 0)
        def _():
            pltpu.sync_copy(stage_v, deg0_hbm.at[pl.ds(r, RCHUNK), :])

        @pl.when(cid == 1)
        def _():
            pltpu.sync_copy(stage_v, deg1_hbm.at[pl.ds(r, RCHUNK), :])


def _degrees(dst1d, ones8, zeros8):
    return pl.kernel(
        _deg_body,
        out_type=[jax.ShapeDtypeStruct((N_PAD, D), jnp.float32)] * 2,
        mesh=_SC_MESH,
        compiler_params=pltpu.CompilerParams(use_tc_tiling_on_sc=False),
        scratch_types=[
            pltpu.VMEM((SUPER,), jnp.int32),
            pltpu.VMEM((SUPER, D), jnp.float32),
            pltpu.VMEM((RCHUNK, D), jnp.float32),
            pltpu.VMEM_SHARED((N_PAD, D), jnp.float32),
            pltpu.SemaphoreType.DMA,
        ],
    )(dst1d, ones8, zeros8)


def _pre_body(deg0_ref, deg1_ref, h0_ref, c1_ref, z_ref, g0_ref, dsq_ref):
    deg = deg0_ref[...] + deg1_ref[...] + 1.0
    dinv = lax.rsqrt(deg)
    h0 = h0_ref[...]
    c1_ref[...] = (1.0 - ALPHA) * dinv * dinv
    z_ref[...] = ALPHA * dinv * h0
    g0_ref[...] = dinv * h0
    dsq_ref[...] = jnp.sqrt(deg)


def _precompute(deg0, deg1, h0p):
    return pl.pallas_call(
        _pre_body,
        grid=(GRID_NPAD,),
        in_specs=[pl.BlockSpec((ROW_BLK, D), lambda i: (i, 0))] * 3,
        out_specs=[pl.BlockSpec((ROW_BLK, D), lambda i: (i, 0))] * 4,
        out_shape=[jax.ShapeDtypeStruct((N_PAD, D), jnp.float32)] * 4,
    )(deg0, deg1, h0p)


def _round_body(src_hbm, dst_hbm, g_hbm, zeros_hbm, s0_hbm, s1_hbm,
                src_v, dst_v, rows_v, stage_v, s_sh,
                gsem0, gsem1, ssem0, ssem1, isem0, isem1, jsem0, jsem1):
    cid = lax.axis_index("c")
    sid = lax.axis_index("s")
    wid = sid * NC + cid
    row0 = sid * ROWS_PER_TILE
    e0 = wid * EDGES_PER_TILE
    gsem = (gsem0, gsem1)
    ssem = (ssem0, ssem1)
    isem = (isem0, isem1)
    jsem = (jsem0, jsem1)
    NS = SUPERS_PER_TILE

    pltpu.sync_copy(zeros_hbm, stage_v)

    @pl.loop(0, NCHUNKS)
    def _zero(c):
        pltpu.sync_copy(stage_v, s_sh.at[pl.ds(row0 + c * RCHUNK, RCHUNK), :])

    plsc.subcore_barrier()

    def src_slice(sc):
        return src_hbm.at[pl.ds(e0 + sc * SUPER, SUPER)]

    def dst_slice(sc):
        return dst_hbm.at[pl.ds(e0 + sc * SUPER, SUPER)]

    def fire_g(sc, p):
        return pltpu.async_copy(g_hbm.at[src_v.at[p]], rows_v.at[p], gsem[p])

    def fire_s(sc, p):
        return pltpu.async_copy(rows_v.at[p], s_sh.at[dst_v.at[p]], ssem[p],
                                add=True)

    def drain_g(p):
        pltpu.make_async_copy(g_hbm.at[src_v.at[p]], rows_v.at[p],
                              gsem[p]).wait()

    def drain_s(p):
        pltpu.make_async_copy(rows_v.at[p], s_sh.at[dst_v.at[p]],
                              ssem[p]).wait()

    pltpu.sync_copy(src_slice(0), src_v.at[0])
    pltpu.sync_copy(dst_slice(0), dst_v.at[0])
    fire_g(0, 0)
    pltpu.async_copy(src_slice(1), src_v.at[1], jsem[1])
    pltpu.async_copy(dst_slice(1), dst_v.at[1], isem[1])

    @pl.loop(0, NS // 2)
    def _pair(t):
        for p in (0, 1):
            sc = t * 2 + p
            q = 1 - p

            if p == 0:
                @pl.when(sc > 0)
                def _():
                    drain_s(q)
                    pltpu.async_copy(dst_slice(sc + 1), dst_v.at[q], isem[q])
            else:
                drain_s(q)
                @pl.when(sc + 1 < NS)
                def _():
                    pltpu.async_copy(dst_slice(sc + 1), dst_v.at[q], isem[q])

            drain_g(p)

            @pl.when(sc + 2 < NS)
            def _():
                pltpu.async_copy(src_slice(sc + 2), src_v.at[p], jsem[p])

            fire_s(sc, p)

            if p == 0:
                pltpu.make_async_copy(src_slice(sc + 1), src_v.at[q],
                                      jsem[q]).wait()
                pltpu.make_async_copy(dst_slice(sc + 1), dst_v.at[q],
                                      isem[q]).wait()
                fire_g(sc + 1, q)
            else:
                @pl.when(sc + 1 < NS)
                def _():
                    pltpu.make_async_copy(src_slice(sc + 1), src_v.at[q],
                                          jsem[q]).wait()
                    pltpu.make_async_copy(dst_slice(sc + 1), dst_v.at[q],
                                          isem[q]).wait()
                    fire_g(sc + 1, q)

    drain_s(1)

    plsc.subcore_barrier()

    @pl.loop(0, NCHUNKS)
    def _dump(c):
        r = row0 + c * RCHUNK
        pltpu.sync_copy(s_sh.at[pl.ds(r, RCHUNK), :], stage_v)

        @pl.when(cid == 0)
        def _():
            pltpu.sync_copy(stage_v, s0_hbm.at[pl.ds(r, RCHUNK), :])

        @pl.when(cid == 1)
        def _():
            pltpu.sync_copy(stage_v, s1_hbm.at[pl.ds(r, RCHUNK), :])


def _scatter_round(src1d, dst1d, g, zeros8):
    return pl.kernel(
        _round_body,
        out_type=[jax.ShapeDtypeStruct((N_PAD, D), jnp.float32)] * 2,
        mesh=_SC_MESH,
        compiler_params=pltpu.CompilerParams(use_tc_tiling_on_sc=False),
        scratch_types=[
            pltpu.VMEM((2, SUPER), jnp.int32),
            pltpu.VMEM((2, SUPER), jnp.int32),
            pltpu.VMEM((2, SUPER, D), jnp.float32),
            pltpu.VMEM((RCHUNK, D), jnp.float32),
            pltpu.VMEM_SHARED((N_PAD, D), jnp.float32),
        ] + [pltpu.SemaphoreType.DMA] * 8,
    )(src1d, dst1d, g, zeros8)


def _dense_body(s0_ref, s1_ref, g_ref, c1_ref, z_ref, o_ref):
    s = s0_ref[...] + s1_ref[...]
    o_ref[...] = c1_ref[...] * (s + g_ref[...]) + z_ref[...]


def _dense_update(s0, s1, g, c1e, z):
    return pl.pallas_call(
        _dense_body,
        grid=(GRID_NPAD,),
        in_specs=[pl.BlockSpec((ROW_BLK, D), lambda i: (i, 0))] * 5,
        out_specs=pl.BlockSpec((ROW_BLK, D), lambda i: (i, 0)),
        out_shape=jax.ShapeDtypeStruct((N_PAD, D), jnp.float32),
    )(s0, s1, g, c1e, z)


def _final_body(g_ref, dsq_ref, o_ref):
    h = g_ref[...] * dsq_ref[...]
    col = lax.broadcasted_iota(jnp.int32, (ROW_BLK, D), 1)
    mask = col < 7
    hm = jnp.where(mask, h, -3e38)
    m = jnp.max(hm, axis=1, keepdims=True)
    e = jnp.where(mask, jnp.exp(h - m), 0.0)
    ls = h - m - jnp.log(jnp.sum(e, axis=1, keepdims=True))
    o_ref[...] = ls[:, :7]


def _finalize(g10, dsq):
    return pl.pallas_call(
        _final_body,
        grid=(GRID_N,),
        in_specs=[pl.BlockSpec((ROW_BLK, D), lambda i: (i, 0))] * 2,
        out_specs=pl.BlockSpec((ROW_BLK, 7), lambda i: (i, 0)),
        out_shape=jax.ShapeDtypeStruct((N, 7), jnp.float32),
    )(g10, dsq)


def kernel(x, edge_index, W1, b1, W2, b2):
    pad = N + jnp.arange(E_PAD - E, dtype=jnp.int32) % (N_PAD - N)
    src1d = jnp.concatenate([edge_index[0], pad])
    dst1d = jnp.concatenate([edge_index[1], pad])

    W2p = jnp.zeros((64, D), jnp.float32).at[:, :7].set(W2)
    b2p = jnp.zeros((1, D), jnp.float32).at[0, :7].set(b2)
    ones8 = jnp.ones((SUPER, D), jnp.float32)
    zeros8 = jnp.zeros((RCHUNK, D), jnp.float32)

    h0 = _mlp(x, W1, b1.reshape(1, 64), W2p, b2p)
    h0p = jnp.pad(h0, ((0, N_PAD - N), (0, 0)))

    deg0, deg1 = _degrees(dst1d, ones8, zeros8)
    c1e, z, g, dsq = _precompute(deg0, deg1, h0p)
    for _ in range(K_ITERS):
        s0, s1 = _scatter_round(src1d, dst1d, g, zeros8)
        g = _dense_update(s0, s1, g, c1e, z)
    return _finalize(g, dsq)
